# baseline (speedup 1.0000x reference)
import functools

import jax
import jax.numpy as jnp
from jax import lax
from jax.experimental import pallas as pl
from jax.experimental.pallas import tpu as pltpu
from jax.experimental.pallas import tpu_sc as plsc

_NC = 2
_NS = 16
_NW = _NC * _NS

_N = 10000
_NP = 10240
_E = 320000
_D = 128
_CH = 100
_NCH = 100
_SB = 10
_NSB = _NCH // _SB
_NBUF = 3
_RPT = _NP // _NS


def _deg_body(col_hbm, out_hbm, colv, zb, onesb, acc, ssem):
    cid = lax.axis_index("c")
    sid = lax.axis_index("s")
    wid = cid * _NS + sid
    pltpu.sync_copy(col_hbm.at[wid], colv)
    for k in range(_RPT // 16):
        zb[pl.ds(k * 16, 16)] = jnp.zeros((16,), jnp.float32)
    for k in range(7):
        onesb[pl.ds(k * 16, 16)] = jnp.full((16,), 1.0, jnp.float32)
    pltpu.sync_copy(zb, acc.at[pl.ds(sid * _RPT, _RPT)])
    plsc.subcore_barrier()

    @pl.loop(0, _NCH)
    def _scatter(j):
        pltpu.async_copy(onesb.at[pl.ds(0, _CH)],
                         acc.at[colv.at[j // _SB, j % _SB]], ssem, add=True)

    @pl.loop(0, _NCH)
    def _drain(j):
        pltpu.make_async_copy(onesb.at[pl.ds(0, _CH)],
                              acc.at[colv.at[j // _SB, j % _SB]], ssem).wait()

    plsc.subcore_barrier()
    pltpu.sync_copy(acc.at[pl.ds(sid * _RPT, _RPT)],
                    out_hbm.at[cid, pl.ds(sid * _RPT, _RPT)])


_deg_call = pl.kernel(
    _deg_body,
    out_type=jax.ShapeDtypeStruct((_NC, _NP), jnp.float32),
    mesh=plsc.VectorSubcoreMesh(core_axis_name="c", subcore_axis_name="s",
                                num_cores=_NC, num_subcores=_NS),
    scratch_types=[
        pltpu.VMEM((_NSB, _SB, _CH), jnp.int32),
        pltpu.VMEM((_RPT,), jnp.float32),
        pltpu.VMEM((112,), jnp.float32),
        pltpu.VMEM_SHARED((_NP,), jnp.float32),
        pltpu.SemaphoreType.DMA,
    ],
)


def _agg_body(row_hbm, col_hbm, p_hbm, out_hbm,
              ra, ca, rb, cb, buf0, buf1, buf2, acc, sem0, sem1, sem2, isem):
    cid = lax.axis_index("c")
    sid = lax.axis_index("s")
    wid = cid * _NS + sid
    @pl.loop(0, _CH, unroll=1)
    def _zero(r):
        for k in range(_D // 16):
            buf2[r, pl.ds(k * 16, 16)] = jnp.zeros((16,), jnp.float32)

    pltpu.sync_copy(row_hbm.at[wid, 0], ra)
    pltpu.sync_copy(col_hbm.at[wid, 0], ca)
    pltpu.async_copy(row_hbm.at[wid, 1], rb, isem)
    pltpu.async_copy(col_hbm.at[wid, 1], cb, isem)

    bufs = (buf0, buf1, buf2)
    sems = (sem0, sem1, sem2)
    rv = (ra, rb)
    cv = (ca, cb)

    def idx_r(j):
        return rv[(j // _SB) % 2].at[j % _SB]

    def idx_c(j):
        return cv[(j // _SB) % 2].at[j % _SB]

    pltpu.async_copy(p_hbm.at[idx_r(0)], buf0, sem0)
    pltpu.async_copy(p_hbm.at[idx_r(1)], buf1, sem1)
    for m in range(_RPT // _CH):
        pltpu.sync_copy(buf2, acc.at[pl.ds(sid * _RPT + m * _CH, _CH)])
    pltpu.sync_copy(buf2.at[pl.ds(0, _RPT % _CH)],
                    acc.at[pl.ds(sid * _RPT + (_RPT // _CH) * _CH,
                                 _RPT % _CH)])
    plsc.subcore_barrier()
    pltpu.async_copy(p_hbm.at[idx_r(2)], buf2, sem2)

    for j in range(_NCH):
        b = j % _NBUF
        sb = j // _SB
        if j % _SB == _SB - _NBUF and j + _NBUF < _NCH:
            nsb = sb + 1
            pltpu.make_async_copy(
                row_hbm.at[wid, nsb], rv[nsb % 2], isem).wait()
            pltpu.make_async_copy(
                col_hbm.at[wid, nsb], cv[nsb % 2], isem).wait()
        pltpu.make_async_copy(p_hbm.at[idx_r(j)], bufs[b], sems[b]).wait()
        pltpu.sync_copy(bufs[b], acc.at[idx_c(j)], add=True)
        if j + _NBUF < _NCH:
            pltpu.async_copy(p_hbm.at[idx_r(j + _NBUF)], bufs[b], sems[b])
        if j % _SB == _SB - 1 and sb + 2 < _NSB:
            pltpu.async_copy(row_hbm.at[wid, sb + 2], rv[sb % 2], isem)
            pltpu.async_copy(col_hbm.at[wid, sb + 2], cv[sb % 2], isem)

    plsc.subcore_barrier()
    pltpu.sync_copy(acc.at[pl.ds(sid * _RPT, _RPT)],
                    out_hbm.at[cid, pl.ds(sid * _RPT, _RPT)])


_agg_call = pl.kernel(
    _agg_body,
    out_type=jax.ShapeDtypeStruct((_NC, _NP, _D), jnp.float32),
    mesh=plsc.VectorSubcoreMesh(core_axis_name="c", subcore_axis_name="s",
                                num_cores=_NC, num_subcores=_NS),
    scratch_types=[
        pltpu.VMEM((_SB, _CH), jnp.int32),
        pltpu.VMEM((_SB, _CH), jnp.int32),
        pltpu.VMEM((_SB, _CH), jnp.int32),
        pltpu.VMEM((_SB, _CH), jnp.int32),
        pltpu.VMEM((_CH, _D), jnp.float32),
        pltpu.VMEM((_CH, _D), jnp.float32),
        pltpu.VMEM((_CH, _D), jnp.float32),
        pltpu.VMEM_SHARED((_NP, _D), jnp.float32),
        pltpu.SemaphoreType.DMA,
        pltpu.SemaphoreType.DMA,
        pltpu.SemaphoreType.DMA,
        pltpu.SemaphoreType.DMA,
    ],
)


_BR = 1024


def _tc0_body(x_ref, w_ref, b_ref, out_ref):
    out_ref[...] = (jnp.dot(x_ref[...], w_ref[...],
                            preferred_element_type=jnp.float32) + b_ref[...])


_tc0_call = pl.pallas_call(
    _tc0_body,
    grid=(_NP // _BR,),
    in_specs=[
        pl.BlockSpec((_BR, _D), lambda i: (i, 0)),
        pl.BlockSpec((_D, _D), lambda i: (0, 0)),
        pl.BlockSpec((1, _D), lambda i: (0, 0)),
    ],
    out_specs=pl.BlockSpec((_BR, _D), lambda i: (i, 0)),
    out_shape=jax.ShapeDtypeStruct((_NP, _D), jnp.float32),
)


def _tc1p_body(x_ref, deg_ref, p_ref, dinv_ref):
    deg = deg_ref[0] + deg_ref[1]
    dinv = jnp.where(deg > 0, lax.rsqrt(deg), 0.0)
    p_ref[...] = x_ref[...] * dinv
    dinv_ref[...] = dinv


_tc1p_call = pl.pallas_call(
    _tc1p_body,
    grid=(_NP // _BR,),
    in_specs=[
        pl.BlockSpec((_BR, _D), lambda i: (i, 0)),
        pl.BlockSpec((_NC, _BR, 1), lambda i: (0, i, 0)),
    ],
    out_specs=[
        pl.BlockSpec((_BR, _D), lambda i: (i, 0)),
        pl.BlockSpec((_BR, 1), lambda i: (i, 0)),
    ],
    out_shape=[
        jax.ShapeDtypeStruct((_NP, _D), jnp.float32),
        jax.ShapeDtypeStruct((_NP, 1), jnp.float32),
    ],
)


def _tc2p_body(s_ref, dv_ref, p_ref):
    dv = dv_ref[...]
    p_ref[...] = (dv * dv) * (s_ref[0] + s_ref[1])


_tc2p_call = pl.pallas_call(
    _tc2p_body,
    grid=(_NP // _BR,),
    in_specs=[
        pl.BlockSpec((_NC, _BR, _D), lambda i: (0, i, 0)),
        pl.BlockSpec((_BR, 1), lambda i: (i, 0)),
    ],
    out_specs=[pl.BlockSpec((_BR, _D), lambda i: (i, 0))],
    out_shape=[jax.ShapeDtypeStruct((_NP, _D), jnp.float32)],
)


def _tc2_body(want_p, s_ref, dv_ref, o_ref, w_ref, out_ref, *maybe_p):
    s = s_ref[0] + s_ref[1]
    dv = dv_ref[...]
    h = dv * s
    out_ref[...] = o_ref[...] + jnp.dot(h, w_ref[...],
                                        preferred_element_type=jnp.float32)
    if want_p:
        maybe_p[0][...] = dv * h


_tc2o_call = pl.pallas_call(
    functools.partial(_tc2_body, False),
    grid=(_NP // _BR,),
    in_specs=[
        pl.BlockSpec((_NC, _BR, _D), lambda i: (0, i, 0)),
        pl.BlockSpec((_BR, 1), lambda i: (i, 0)),
        pl.BlockSpec((_BR, _D), lambda i: (i, 0)),
        pl.BlockSpec((_D, _D), lambda i: (0, 0)),
    ],
    out_specs=[pl.BlockSpec((_BR, _D), lambda i: (i, 0))],
    out_shape=[jax.ShapeDtypeStruct((_NP, _D), jnp.float32)],
)

_BF = 2000
_tc2_last_call = pl.pallas_call(
    functools.partial(_tc2_body, False),
    grid=(_N // _BF,),
    in_specs=[
        pl.BlockSpec((_NC, _BF, _D), lambda i: (0, i, 0)),
        pl.BlockSpec((_BF, 1), lambda i: (i, 0)),
        pl.BlockSpec((_BF, _D), lambda i: (i, 0)),
        pl.BlockSpec((_D, _D), lambda i: (0, 0)),
    ],
    out_specs=[pl.BlockSpec((_BF, _D), lambda i: (i, 0))],
    out_shape=[jax.ShapeDtypeStruct((_N, _D), jnp.float32)],
)


def kernel(x, edge_index, W0, W1, W2, W3, bias):
    row = edge_index[0].reshape(_NW, _NSB, _SB, _CH)
    col = edge_index[1].reshape(_NW, _NSB, _SB, _CH)
    x_pad = jnp.pad(x, ((0, _NP - _N), (0, 0)))

    out = _tc0_call(x_pad, W0, bias.reshape(1, _D))
    degp = _deg_call(col)
    p, dinv = _tc1p_call(x_pad, degp.reshape(_NC, _NP, 1))
    for W in (W1, W2):
        sp = _agg_call(row, col, p)
        (p,) = _tc2p_call(sp, dinv)
        (out,) = _tc2o_call(sp, dinv, out, W)
    sp = _agg_call(row, col, p)
    (out,) = _tc2_last_call(sp, dinv, out, W3)
    return (out, edge_index)


# speedup vs baseline: 1.0132x; 1.0132x over previous
import functools

import jax
import jax.numpy as jnp
from jax import lax
from jax.experimental import pallas as pl
from jax.experimental.pallas import tpu as pltpu
from jax.experimental.pallas import tpu_sc as plsc

_NC = 2
_NS = 16
_NW = _NC * _NS

_N = 10000
_NP = 10240
_E = 320000
_D = 128
_CH = 100
_NCH = 100
_SB = 10
_NSB = _NCH // _SB
_NBUF = 3
_RPT = _NP // _NS


def _deg_body(col_hbm, out_hbm, colv, zb, onesb, acc, ssem):
    cid = lax.axis_index("c")
    sid = lax.axis_index("s")
    wid = cid * _NS + sid
    pltpu.sync_copy(col_hbm.at[wid], colv)
    for k in range(_RPT // 16):
        zb[pl.ds(k * 16, 16)] = jnp.zeros((16,), jnp.float32)
    for k in range(7):
        onesb[pl.ds(k * 16, 16)] = jnp.full((16,), 1.0, jnp.float32)
    pltpu.sync_copy(zb, acc.at[pl.ds(sid * _RPT, _RPT)])
    plsc.subcore_barrier()

    @pl.loop(0, _NCH)
    def _scatter(j):
        pltpu.async_copy(onesb.at[pl.ds(0, _CH)],
                         acc.at[colv.at[j // _SB, j % _SB]], ssem, add=True)

    @pl.loop(0, _NCH)
    def _drain(j):
        pltpu.make_async_copy(onesb.at[pl.ds(0, _CH)],
                              acc.at[colv.at[j // _SB, j % _SB]], ssem).wait()

    plsc.subcore_barrier()
    pltpu.sync_copy(acc.at[pl.ds(sid * _RPT, _RPT)],
                    out_hbm.at[cid, pl.ds(sid * _RPT, _RPT)])


_deg_call = pl.kernel(
    _deg_body,
    out_type=jax.ShapeDtypeStruct((_NC, _NP), jnp.float32),
    mesh=plsc.VectorSubcoreMesh(core_axis_name="c", subcore_axis_name="s",
                                num_cores=_NC, num_subcores=_NS),
    scratch_types=[
        pltpu.VMEM((_NSB, _SB, _CH), jnp.int32),
        pltpu.VMEM((_RPT,), jnp.float32),
        pltpu.VMEM((112,), jnp.float32),
        pltpu.VMEM_SHARED((_NP,), jnp.float32),
        pltpu.SemaphoreType.DMA,
    ],
)


def _agg_body(row_hbm, col_hbm, p_hbm, out_hbm,
              ra, ca, rb, cb, buf0, buf1, buf2, acc, sem0, sem1, sem2, isem):
    cid = lax.axis_index("c")
    sid = lax.axis_index("s")
    wid = cid * _NS + sid
    @pl.loop(0, _CH, unroll=1)
    def _zero(r):
        for k in range(_D // 16):
            buf2[r, pl.ds(k * 16, 16)] = jnp.zeros((16,), jnp.float32)

    pltpu.sync_copy(row_hbm.at[wid, 0], ra)
    pltpu.sync_copy(col_hbm.at[wid, 0], ca)
    pltpu.async_copy(row_hbm.at[wid, 1], rb, isem)
    pltpu.async_copy(col_hbm.at[wid, 1], cb, isem)

    bufs = (buf0, buf1, buf2)
    sems = (sem0, sem1, sem2)
    rv = (ra, rb)
    cv = (ca, cb)

    def idx_r(j):
        return rv[(j // _SB) % 2].at[j % _SB]

    def idx_c(j):
        return cv[(j // _SB) % 2].at[j % _SB]

    pltpu.async_copy(p_hbm.at[idx_r(0)], buf0, sem0)
    pltpu.async_copy(p_hbm.at[idx_r(1)], buf1, sem1)
    for m in range(_RPT // _CH):
        pltpu.sync_copy(buf2, acc.at[pl.ds(sid * _RPT + m * _CH, _CH)])
    pltpu.sync_copy(buf2.at[pl.ds(0, _RPT % _CH)],
                    acc.at[pl.ds(sid * _RPT + (_RPT // _CH) * _CH,
                                 _RPT % _CH)])
    plsc.subcore_barrier()
    pltpu.async_copy(p_hbm.at[idx_r(2)], buf2, sem2)

    for j in range(_NCH):
        b = j % _NBUF
        sb = j // _SB
        if j % _SB == _SB - _NBUF and j + _NBUF < _NCH:
            nsb = sb + 1
            pltpu.make_async_copy(
                row_hbm.at[wid, nsb], rv[nsb % 2], isem).wait()
            pltpu.make_async_copy(
                col_hbm.at[wid, nsb], cv[nsb % 2], isem).wait()
        pltpu.make_async_copy(p_hbm.at[idx_r(j)], bufs[b], sems[b]).wait()
        pltpu.sync_copy(bufs[b], acc.at[idx_c(j)], add=True)
        if j + _NBUF < _NCH:
            pltpu.async_copy(p_hbm.at[idx_r(j + _NBUF)], bufs[b], sems[b])
        if j % _SB == _SB - 1 and sb + 2 < _NSB:
            pltpu.async_copy(row_hbm.at[wid, sb + 2], rv[sb % 2], isem)
            pltpu.async_copy(col_hbm.at[wid, sb + 2], cv[sb % 2], isem)

    plsc.subcore_barrier()
    pltpu.sync_copy(acc.at[pl.ds(sid * _RPT, _RPT)],
                    out_hbm.at[cid, pl.ds(sid * _RPT, _RPT)])


_agg_call = pl.kernel(
    _agg_body,
    out_type=jax.ShapeDtypeStruct((_NC, _NP, _D), jnp.float32),
    mesh=plsc.VectorSubcoreMesh(core_axis_name="c", subcore_axis_name="s",
                                num_cores=_NC, num_subcores=_NS),
    scratch_types=[
        pltpu.VMEM((_SB, _CH), jnp.int32),
        pltpu.VMEM((_SB, _CH), jnp.int32),
        pltpu.VMEM((_SB, _CH), jnp.int32),
        pltpu.VMEM((_SB, _CH), jnp.int32),
        pltpu.VMEM((_CH, _D), jnp.float32),
        pltpu.VMEM((_CH, _D), jnp.float32),
        pltpu.VMEM((_CH, _D), jnp.float32),
        pltpu.VMEM_SHARED((_NP, _D), jnp.float32),
        pltpu.SemaphoreType.DMA,
        pltpu.SemaphoreType.DMA,
        pltpu.SemaphoreType.DMA,
        pltpu.SemaphoreType.DMA,
    ],
)


_BR = 1024


def _tc0_body(x_ref, w_ref, b_ref, out_ref):
    out_ref[...] = (jnp.dot(x_ref[...], w_ref[...],
                            preferred_element_type=jnp.float32) + b_ref[...])


_tc0_call = pl.pallas_call(
    _tc0_body,
    grid=(_NP // _BR,),
    in_specs=[
        pl.BlockSpec((_BR, _D), lambda i: (i, 0)),
        pl.BlockSpec((_D, _D), lambda i: (0, 0)),
        pl.BlockSpec((1, _D), lambda i: (0, 0)),
    ],
    out_specs=pl.BlockSpec((_BR, _D), lambda i: (i, 0)),
    out_shape=jax.ShapeDtypeStruct((_NP, _D), jnp.float32),
)


def _tc1p_body(x_ref, deg_ref, p_ref, dinv_ref):
    deg = deg_ref[0] + deg_ref[1]
    dinv = jnp.where(deg > 0, lax.rsqrt(deg), 0.0)
    p_ref[...] = x_ref[...] * dinv
    dinv_ref[...] = dinv


_tc1p_call = pl.pallas_call(
    _tc1p_body,
    grid=(_NP // _BR,),
    in_specs=[
        pl.BlockSpec((_BR, _D), lambda i: (i, 0)),
        pl.BlockSpec((_NC, _BR, 1), lambda i: (0, i, 0)),
    ],
    out_specs=[
        pl.BlockSpec((_BR, _D), lambda i: (i, 0)),
        pl.BlockSpec((_BR, 1), lambda i: (i, 0)),
    ],
    out_shape=[
        jax.ShapeDtypeStruct((_NP, _D), jnp.float32),
        jax.ShapeDtypeStruct((_NP, 1), jnp.float32),
    ],
)


def _tc2p_body(s_ref, dv_ref, p_ref):
    dv = dv_ref[...]
    p_ref[...] = (dv * dv) * (s_ref[0] + s_ref[1])


_tc2p_call = pl.pallas_call(
    _tc2p_body,
    grid=(_NP // _BR,),
    in_specs=[
        pl.BlockSpec((_NC, _BR, _D), lambda i: (0, i, 0)),
        pl.BlockSpec((_BR, 1), lambda i: (i, 0)),
    ],
    out_specs=[pl.BlockSpec((_BR, _D), lambda i: (i, 0))],
    out_shape=[jax.ShapeDtypeStruct((_NP, _D), jnp.float32)],
)


def _tc2_body(want_p, s_ref, dv_ref, o_ref, w_ref, out_ref, *maybe_p):
    s = s_ref[0] + s_ref[1]
    dv = dv_ref[...]
    h = dv * s
    out_ref[...] = o_ref[...] + jnp.dot(h, w_ref[...],
                                        preferred_element_type=jnp.float32)
    if want_p:
        maybe_p[0][...] = dv * h


_tc2_call = pl.pallas_call(
    functools.partial(_tc2_body, True),
    grid=(_NP // _BR,),
    in_specs=[
        pl.BlockSpec((_NC, _BR, _D), lambda i: (0, i, 0)),
        pl.BlockSpec((_BR, 1), lambda i: (i, 0)),
        pl.BlockSpec((_BR, _D), lambda i: (i, 0)),
        pl.BlockSpec((_D, _D), lambda i: (0, 0)),
    ],
    out_specs=[pl.BlockSpec((_BR, _D), lambda i: (i, 0))] * 2,
    out_shape=[jax.ShapeDtypeStruct((_NP, _D), jnp.float32)] * 2,
)

_BF = 2000
_tc2_last_call = pl.pallas_call(
    functools.partial(_tc2_body, False),
    grid=(_N // _BF,),
    in_specs=[
        pl.BlockSpec((_NC, _BF, _D), lambda i: (0, i, 0)),
        pl.BlockSpec((_BF, 1), lambda i: (i, 0)),
        pl.BlockSpec((_BF, _D), lambda i: (i, 0)),
        pl.BlockSpec((_D, _D), lambda i: (0, 0)),
    ],
    out_specs=[pl.BlockSpec((_BF, _D), lambda i: (i, 0))],
    out_shape=[jax.ShapeDtypeStruct((_N, _D), jnp.float32)],
)


def kernel(x, edge_index, W0, W1, W2, W3, bias):
    row = edge_index[0].reshape(_NW, _NSB, _SB, _CH)
    col = edge_index[1].reshape(_NW, _NSB, _SB, _CH)
    x_pad = jnp.pad(x, ((0, _NP - _N), (0, 0)))

    out = _tc0_call(x_pad, W0, bias.reshape(1, _D))
    degp = _deg_call(col)
    p, dinv = _tc1p_call(x_pad, degp.reshape(_NC, _NP, 1))
    for W in (W1, W2):
        sp = _agg_call(row, col, p)
        out, p = _tc2_call(sp, dinv, out, W)
    sp = _agg_call(row, col, p)
    (out,) = _tc2_last_call(sp, dinv, out, W3)
    return (out, edge_index)


# speedup vs baseline: 1.0136x; 1.0004x over previous
import functools

import jax
import jax.numpy as jnp
from jax import lax
from jax.experimental import pallas as pl
from jax.experimental.pallas import tpu as pltpu
from jax.experimental.pallas import tpu_sc as plsc

_NC = 2
_NS = 16
_NW = _NC * _NS

_N = 10000
_NP = 10240
_E = 320000
_D = 128
_CH = 100
_NCH = 100
_SB = 10
_NSB = _NCH // _SB
_NBUF = 3
_RPT = _NP // _NS


def _deg_body(col_hbm, out_hbm, colv, zb, onesb, acc, ssem):
    cid = lax.axis_index("c")
    sid = lax.axis_index("s")
    wid = cid * _NS + sid
    pltpu.sync_copy(col_hbm.at[wid], colv)
    for k in range(_RPT // 16):
        zb[pl.ds(k * 16, 16)] = jnp.zeros((16,), jnp.float32)
    for k in range(7):
        onesb[pl.ds(k * 16, 16)] = jnp.full((16,), 1.0, jnp.float32)
    pltpu.sync_copy(zb, acc.at[pl.ds(sid * _RPT, _RPT)])
    plsc.subcore_barrier()

    @pl.loop(0, _NCH)
    def _scatter(j):
        pltpu.async_copy(onesb.at[pl.ds(0, _CH)],
                         acc.at[colv.at[j // _SB, j % _SB]], ssem, add=True)

    @pl.loop(0, _NCH)
    def _drain(j):
        pltpu.make_async_copy(onesb.at[pl.ds(0, _CH)],
                              acc.at[colv.at[j // _SB, j % _SB]], ssem).wait()

    plsc.subcore_barrier()
    pltpu.sync_copy(acc.at[pl.ds(sid * _RPT, _RPT)],
                    out_hbm.at[cid, pl.ds(sid * _RPT, _RPT)])


_deg_call = pl.kernel(
    _deg_body,
    out_type=jax.ShapeDtypeStruct((_NC, _NP), jnp.float32),
    mesh=plsc.VectorSubcoreMesh(core_axis_name="c", subcore_axis_name="s",
                                num_cores=_NC, num_subcores=_NS),
    scratch_types=[
        pltpu.VMEM((_NSB, _SB, _CH), jnp.int32),
        pltpu.VMEM((_RPT,), jnp.float32),
        pltpu.VMEM((112,), jnp.float32),
        pltpu.VMEM_SHARED((_NP,), jnp.float32),
        pltpu.SemaphoreType.DMA,
    ],
)


def _agg_body(row_hbm, col_hbm, p_hbm, out_hbm,
              ra, ca, rb, cb, buf0, buf1, buf2, acc, sem0, sem1, sem2, isem):
    cid = lax.axis_index("c")
    sid = lax.axis_index("s")
    wid = cid * _NS + sid
    @pl.loop(0, _CH, unroll=1)
    def _zero(r):
        for k in range(_D // 16):
            buf2[r, pl.ds(k * 16, 16)] = jnp.zeros((16,), jnp.float32)

    pltpu.sync_copy(row_hbm.at[wid, 0], ra)
    pltpu.sync_copy(col_hbm.at[wid, 0], ca)
    pltpu.async_copy(row_hbm.at[wid, 1], rb, isem)
    pltpu.async_copy(col_hbm.at[wid, 1], cb, isem)

    bufs = (buf0, buf1, buf2)
    sems = (sem0, sem1, sem2)
    rv = (ra, rb)
    cv = (ca, cb)

    def idx_r(j):
        return rv[(j // _SB) % 2].at[j % _SB]

    def idx_c(j):
        return cv[(j // _SB) % 2].at[j % _SB]

    pltpu.async_copy(p_hbm.at[idx_r(0)], buf0, sem0)
    pltpu.async_copy(p_hbm.at[idx_r(1)], buf1, sem1)
    for m in range(_RPT // _CH):
        pltpu.sync_copy(buf2, acc.at[pl.ds(sid * _RPT + m * _CH, _CH)])
    pltpu.sync_copy(buf2.at[pl.ds(0, _RPT % _CH)],
                    acc.at[pl.ds(sid * _RPT + (_RPT // _CH) * _CH,
                                 _RPT % _CH)])
    plsc.subcore_barrier()
    pltpu.async_copy(p_hbm.at[idx_r(2)], buf2, sem2)

    for j in range(_NCH):
        b = j % _NBUF
        sb = j // _SB
        if j % _SB == _SB - _NBUF and j + _NBUF < _NCH:
            nsb = sb + 1
            pltpu.make_async_copy(
                row_hbm.at[wid, nsb], rv[nsb % 2], isem).wait()
            pltpu.make_async_copy(
                col_hbm.at[wid, nsb], cv[nsb % 2], isem).wait()
        pltpu.make_async_copy(p_hbm.at[idx_r(j)], bufs[b], sems[b]).wait()
        pltpu.sync_copy(bufs[b], acc.at[idx_c(j)], add=True)
        if j + _NBUF < _NCH:
            pltpu.async_copy(p_hbm.at[idx_r(j + _NBUF)], bufs[b], sems[b])
        if j % _SB == _SB - 1 and sb + 2 < _NSB:
            pltpu.async_copy(row_hbm.at[wid, sb + 2], rv[sb % 2], isem)
            pltpu.async_copy(col_hbm.at[wid, sb + 2], cv[sb % 2], isem)

    plsc.subcore_barrier()
    pltpu.sync_copy(acc.at[pl.ds(sid * _RPT, _RPT)],
                    out_hbm.at[cid, pl.ds(sid * _RPT, _RPT)])


_agg_call = pl.kernel(
    _agg_body,
    out_type=jax.ShapeDtypeStruct((_NC, _NP, _D), jnp.float32),
    mesh=plsc.VectorSubcoreMesh(core_axis_name="c", subcore_axis_name="s",
                                num_cores=_NC, num_subcores=_NS),
    scratch_types=[
        pltpu.VMEM((_SB, _CH), jnp.int32),
        pltpu.VMEM((_SB, _CH), jnp.int32),
        pltpu.VMEM((_SB, _CH), jnp.int32),
        pltpu.VMEM((_SB, _CH), jnp.int32),
        pltpu.VMEM((_CH, _D), jnp.float32),
        pltpu.VMEM((_CH, _D), jnp.float32),
        pltpu.VMEM((_CH, _D), jnp.float32),
        pltpu.VMEM_SHARED((_NP, _D), jnp.float32),
        pltpu.SemaphoreType.DMA,
        pltpu.SemaphoreType.DMA,
        pltpu.SemaphoreType.DMA,
        pltpu.SemaphoreType.DMA,
    ],
)


_BR = 1024


def _tc0_body(x_ref, w_ref, b_ref, out_ref):
    out_ref[...] = (jnp.dot(x_ref[...], w_ref[...],
                            preferred_element_type=jnp.float32) + b_ref[...])


_tc0_call = pl.pallas_call(
    _tc0_body,
    grid=(_NP // _BR,),
    in_specs=[
        pl.BlockSpec((_BR, _D), lambda i: (i, 0)),
        pl.BlockSpec((_D, _D), lambda i: (0, 0)),
        pl.BlockSpec((1, _D), lambda i: (0, 0)),
    ],
    out_specs=pl.BlockSpec((_BR, _D), lambda i: (i, 0)),
    out_shape=jax.ShapeDtypeStruct((_NP, _D), jnp.float32),
)


def _tc1p_body(x_ref, deg_ref, p_ref, dinv_ref):
    deg = deg_ref[0] + deg_ref[1]
    dinv = jnp.where(deg > 0, lax.rsqrt(deg), 0.0)
    p_ref[...] = x_ref[...] * dinv
    dinv_ref[...] = dinv


_tc1p_call = pl.pallas_call(
    _tc1p_body,
    grid=(_NP // _BR,),
    in_specs=[
        pl.BlockSpec((_BR, _D), lambda i: (i, 0)),
        pl.BlockSpec((_NC, _BR, 1), lambda i: (0, i, 0)),
    ],
    out_specs=[
        pl.BlockSpec((_BR, _D), lambda i: (i, 0)),
        pl.BlockSpec((_BR, 1), lambda i: (i, 0)),
    ],
    out_shape=[
        jax.ShapeDtypeStruct((_NP, _D), jnp.float32),
        jax.ShapeDtypeStruct((_NP, 1), jnp.float32),
    ],
)


def _tc2_body(want_p, s_ref, dv_ref, o_ref, w_ref, out_ref, *maybe_p):
    s = s_ref[0] + s_ref[1]
    dv = dv_ref[...]
    h = dv * s
    out_ref[...] = o_ref[...] + jnp.dot(h, w_ref[...],
                                        preferred_element_type=jnp.float32)
    if want_p:
        maybe_p[0][...] = dv * h


_tc2_call = pl.pallas_call(
    functools.partial(_tc2_body, True),
    grid=(_NP // _BR,),
    in_specs=[
        pl.BlockSpec((_NC, _BR, _D), lambda i: (0, i, 0)),
        pl.BlockSpec((_BR, 1), lambda i: (i, 0)),
        pl.BlockSpec((_BR, _D), lambda i: (i, 0)),
        pl.BlockSpec((_D, _D), lambda i: (0, 0)),
    ],
    out_specs=[pl.BlockSpec((_BR, _D), lambda i: (i, 0))] * 2,
    out_shape=[jax.ShapeDtypeStruct((_NP, _D), jnp.float32)] * 2,
)

_BF = 2000
_tc2_last_call = pl.pallas_call(
    functools.partial(_tc2_body, False),
    grid=(_N // _BF,),
    in_specs=[
        pl.BlockSpec((_NC, _BF, _D), lambda i: (0, i, 0)),
        pl.BlockSpec((_BF, 1), lambda i: (i, 0)),
        pl.BlockSpec((_BF, _D), lambda i: (i, 0)),
        pl.BlockSpec((_D, _D), lambda i: (0, 0)),
    ],
    out_specs=[pl.BlockSpec((_BF, _D), lambda i: (i, 0))],
    out_shape=[jax.ShapeDtypeStruct((_N, _D), jnp.float32)],
)


def kernel(x, edge_index, W0, W1, W2, W3, bias):
    row = edge_index[0].reshape(_NW, _NSB, _SB, _CH)
    col = edge_index[1].reshape(_NW, _NSB, _SB, _CH)
    x_pad = jnp.pad(x, ((0, _NP - _N), (0, 0)))

    out = _tc0_call(x_pad, W0, bias.reshape(1, _D))
    degp = _deg_call(col)
    p, dinv = _tc1p_call(x_pad, degp.reshape(_NC, _NP, 1))
    for W in (W1, W2):
        sp = _agg_call(row, col, p)
        out, p = _tc2_call(sp, dinv, out, W)
    sp = _agg_call(row, col, p)
    (out,) = _tc2_last_call(sp, dinv, out, W3)
    return (out, edge_index)
